# baseline (speedup 1.0000x reference)
import functools

import jax
import jax.numpy as jnp
from jax import lax
from jax.experimental import pallas as pl
from jax.experimental.pallas import tpu as pltpu
from jax.experimental.pallas import tpu_sc as plsc

_L = 16


@functools.lru_cache(maxsize=None)
def _build(B, D, K):
    NC, NS = 2, 16
    NW = NC * NS
    RPW = B // NW
    CH = 64
    NCHUNK = RPW // CH
    NJ = K // _L

    mesh = plsc.VectorSubcoreMesh(core_axis_name="c", subcore_axis_name="s")

    @functools.partial(
        pl.kernel,
        mesh=mesh,
        compiler_params=pltpu.CompilerParams(needs_layout_passes=False),
        out_type=jax.ShapeDtypeStruct((2 * B * K,), jnp.float32),
        scratch_types=[
            pltpu.VMEM((K,), jnp.int32),
            pltpu.VMEM((CH * D,), jnp.float32),
            pltpu.VMEM((CH * D,), jnp.float32),
            pltpu.VMEM((CH * K,), jnp.float32),
            pltpu.VMEM((CH * K,), jnp.float32),
        ],
    )
    def sc_kernel(xc_hbm, xd_hbm, idx_hbm, out_hbm, idx_v, c_v, d_v, l_v, r_v):
        wid = lax.axis_index("s") * NC + lax.axis_index("c")
        base = wid * RPW
        pltpu.sync_copy(idx_hbm, idx_v)
        cols = [idx_v[pl.ds(j * _L, _L)] for j in range(NJ)]

        def chunk_body(ch, carry):
            row0 = base + ch * CH
            pltpu.sync_copy(xc_hbm.at[pl.ds(row0 * D, CH * D)], c_v)
            pltpu.sync_copy(xd_hbm.at[pl.ds(row0 * D, CH * D)], d_v)

            def row_body(r, carry2):
                rbase = r * D
                obase = r * K
                for j in range(NJ):
                    idx = cols[j] + rbase
                    gc = plsc.load_gather(c_v, [idx])
                    gd = plsc.load_gather(d_v, [idx])
                    l_v[pl.ds(obase + j * _L, _L)] = gc - gd
                    r_v[pl.ds(obase + j * _L, _L)] = gc + gd
                return carry2

            lax.fori_loop(0, CH, row_body, 0)
            pltpu.sync_copy(l_v, out_hbm.at[pl.ds(row0 * K, CH * K)])
            pltpu.sync_copy(r_v, out_hbm.at[pl.ds(B * K + row0 * K, CH * K)])
            return carry

        lax.fori_loop(0, NCHUNK, chunk_body, 0)

    return sc_kernel


def kernel(x_c, x_delta, target_idx):
    B, D = x_c.shape
    (K,) = target_idx.shape
    sc_kernel = _build(B, D, K)
    out = sc_kernel(
        x_c.reshape(B * D),
        x_delta.reshape(B * D),
        target_idx.astype(jnp.int32),
    )
    return out.reshape(2, B, K)


# speedup vs baseline: 1.3422x; 1.3422x over previous
import functools

import jax
import jax.numpy as jnp
from jax import lax
from jax.experimental import pallas as pl
from jax.experimental.pallas import tpu as pltpu
from jax.experimental.pallas import tpu_sc as plsc

_L = 16


@functools.lru_cache(maxsize=None)
def _build(B, D, K):
    NC, NS = 2, 16
    NW = NC * NS
    RPW = B // NW
    CH = 128
    NCHUNK = RPW // CH
    NJ = K // _L

    mesh = plsc.VectorSubcoreMesh(core_axis_name="c", subcore_axis_name="s")

    @functools.partial(
        pl.kernel,
        mesh=mesh,
        compiler_params=pltpu.CompilerParams(needs_layout_passes=False),
        out_type=jax.ShapeDtypeStruct((2 * B * K,), jnp.float32),
        scratch_types=[
            pltpu.VMEM((K,), jnp.int32),
            pltpu.VMEM((CH * D,), jnp.float32),
            pltpu.VMEM((CH * D,), jnp.float32),
            pltpu.VMEM((CH * D,), jnp.float32),
            pltpu.VMEM((CH * D,), jnp.float32),
            pltpu.VMEM((CH * K,), jnp.float32),
            pltpu.VMEM((CH * K,), jnp.float32),
            pltpu.VMEM((CH * K,), jnp.float32),
            pltpu.VMEM((CH * K,), jnp.float32),
            pltpu.SemaphoreType.DMA,
            pltpu.SemaphoreType.DMA,
            pltpu.SemaphoreType.DMA,
            pltpu.SemaphoreType.DMA,
        ],
    )
    def sc_kernel(xc_hbm, xd_hbm, idx_hbm, out_hbm,
                  idx_v, c_v0, c_v1, d_v0, d_v1, l_v0, l_v1, r_v0, r_v1,
                  in_sem0, in_sem1, out_sem0, out_sem1):
        c_v = [c_v0, c_v1]
        d_v = [d_v0, d_v1]
        l_v = [l_v0, l_v1]
        r_v = [r_v0, r_v1]
        in_sem = [in_sem0, in_sem1]
        out_sem = [out_sem0, out_sem1]
        wid = lax.axis_index("s") * NC + lax.axis_index("c")
        base = wid * RPW
        pltpu.sync_copy(idx_hbm, idx_v)
        cols = [idx_v[pl.ds(j * _L, _L)] for j in range(NJ)]

        def start_in(ch, b):
            row0 = base + ch * CH
            cc = pltpu.async_copy(
                xc_hbm.at[pl.ds(row0 * D, CH * D)], c_v[b], in_sem[b])
            dc = pltpu.async_copy(
                xd_hbm.at[pl.ds(row0 * D, CH * D)], d_v[b], in_sem[b])
            return cc, dc

        def start_out(ch, b):
            row0 = base + ch * CH
            lc = pltpu.async_copy(
                l_v[b], out_hbm.at[pl.ds(row0 * K, CH * K)], out_sem[b])
            rc = pltpu.async_copy(
                r_v[b],
                out_hbm.at[pl.ds(B * K + row0 * K, CH * K)], out_sem[b])
            return lc, rc

        def compute(b):
            lb = l_v[b]
            rb = r_v[b]
            cb = c_v[b]
            db = d_v[b]

            @plsc.parallel_loop(0, CH, unroll=4)
            def row_body(r):
                rbase = r * D
                obase = r * K
                for j in range(NJ):
                    idx = cols[j] + rbase
                    gc = plsc.load_gather(cb, [idx])
                    gd = plsc.load_gather(db, [idx])
                    lb[pl.ds(obase + j * _L, _L)] = gc - gd
                    rb[pl.ds(obase + j * _L, _L)] = gc + gd

        in_flight = start_in(0, 0)
        out_flight = [None, None]
        for ch in range(NCHUNK):
            b = ch % 2
            nxt = None
            if ch + 1 < NCHUNK:
                nxt = start_in(ch + 1, 1 - b)
            in_flight[0].wait()
            in_flight[1].wait()
            if out_flight[b] is not None:
                out_flight[b][0].wait()
                out_flight[b][1].wait()
            compute(b)
            out_flight[b] = start_out(ch, b)
            in_flight = nxt
        for b in range(2):
            if out_flight[b] is not None:
                out_flight[b][0].wait()
                out_flight[b][1].wait()

    return sc_kernel


def kernel(x_c, x_delta, target_idx):
    B, D = x_c.shape
    (K,) = target_idx.shape
    sc_kernel = _build(B, D, K)
    out = sc_kernel(
        x_c.reshape(B * D),
        x_delta.reshape(B * D),
        target_idx.astype(jnp.int32),
    )
    return out.reshape(2, B, K)
